# Initial kernel scaffold; baseline (speedup 1.0000x reference)
import jax
import jax.numpy as jnp
from jax.experimental import pallas as pl


def kernel(atom_types, pos, edge_index, timesteps, batch, params):
    raise NotImplementedError("write your pallas kernel here")


# speedup vs baseline: 1.9179x; 1.9179x over previous
import functools
import math

import jax
import jax.numpy as jnp
from jax import lax
from jax.experimental import pallas as pl
from jax.experimental.pallas import tpu as pltpu
from jax.experimental.pallas import tpu_sc as plsc

NN, NE, HD, NB, NL, NA = 10000, 320000, 128, 128, 8, 10
NC, NS = 2, 16
NW = NC * NS
EPW = NE // NW
CH = 200
NCHUNK = EPW // CH
G16 = CH // 16
NPT = (-(-NN // NW) + 7) // 8 * 8
NPT_LAST = NN - (NW - 1) * NPT


@functools.cache
def _sc_mesh():
  return plsc.VectorSubcoreMesh(
      core_axis_name="c", subcore_axis_name="s", num_cores=NC, num_subcores=NS)


def _rsqrt16(x):
  i = plsc.bitcast(x, jnp.int32)
  i = jnp.int32(0x5F3759DF) - lax.shift_right_logical(i, 1)
  y = plsc.bitcast(i, jnp.float32)
  for _ in range(3):
    y = y * (1.5 - 0.5 * x * y * y)
  return y


def _sc_gather_body(a_tab, b_tab, pos4, row, col,
                    ar_out, br_out, du_out,
                    idxr, idxc, abuf, bbuf, pos_v, dubuf, sem0, sem1):
  c = lax.axis_index("c")
  s = lax.axis_index("s")
  base = (s * NC + c) * EPW

  pltpu.sync_copy(pos4, pos_v)
  zero16 = jnp.zeros((16,), jnp.int32)
  idxr[pl.ds(CH - 8, 16)] = zero16
  idxc[pl.ds(CH - 8, 16)] = zero16

  def chunk(k, carry):
    off = base + k * CH
    pltpu.sync_copy(row.at[pl.ds(off, CH)], idxr.at[pl.ds(0, CH)])
    pltpu.sync_copy(col.at[pl.ds(off, CH)], idxc.at[pl.ds(0, CH)])
    cp0 = pltpu.async_copy(a_tab.at[idxr.at[pl.ds(0, CH)]], abuf, sem0)
    cp1 = pltpu.async_copy(b_tab.at[idxc.at[pl.ds(0, CH)]], bbuf, sem1)

    lanes = lax.iota(jnp.int32, 16)
    for g in range(G16 + 1):
      mask = None if g < G16 else lanes < (CH - G16 * 16)
      r16 = idxr[pl.ds(g * 16, 16)]
      c16 = idxc[pl.ds(g * 16, 16)]
      d = []
      for comp in range(3):
        pr = plsc.load_gather(pos_v, [r16 * 4 + comp], mask=mask)
        qr = plsc.load_gather(pos_v, [c16 * 4 + comp], mask=mask)
        d.append(pr - qr)
      d2 = d[0] * d[0] + d[1] * d[1] + d[2] * d[2] + 1e-8
      y = _rsqrt16(d2)
      dist = d2 * y
      inv = 1.0 / (dist + 1e-8)
      e16 = g * 16 + lanes
      for comp in range(3):
        plsc.store_scatter(dubuf, [e16 * 4 + comp], d[comp] * inv, mask=mask)
      plsc.store_scatter(dubuf, [e16 * 4 + 3], dist, mask=mask)

    cp0.wait()
    cp1.wait()
    pltpu.sync_copy(abuf, ar_out.at[pl.ds(off, CH)])
    pltpu.sync_copy(bbuf, br_out.at[pl.ds(off, CH)])
    pltpu.sync_copy(dubuf, du_out.at[pl.ds(off * 4, CH * 4)])
    return carry

  lax.fori_loop(0, NCHUNK, chunk, 0)


@functools.cache
def _sc_gather():
  return pl.kernel(
      _sc_gather_body,
      out_type=(
          jax.ShapeDtypeStruct((NE, HD), jnp.float32),
          jax.ShapeDtypeStruct((NE, HD), jnp.float32),
          jax.ShapeDtypeStruct((NE * 4,), jnp.float32),
      ),
      mesh=_sc_mesh(),
      compiler_params=pltpu.CompilerParams(needs_layout_passes=False),
      scratch_types=[
          pltpu.VMEM((CH + 8,), jnp.int32),
          pltpu.VMEM((CH + 8,), jnp.int32),
          pltpu.VMEM((CH, HD), jnp.float32),
          pltpu.VMEM((CH, HD), jnp.float32),
          pltpu.VMEM((NN * 4,), jnp.float32),
          pltpu.VMEM((CH * 4,), jnp.float32),
          pltpu.SemaphoreType.DMA,
          pltpu.SemaphoreType.DMA,
      ],
  )


def _sc_scatter_body(m, cwu, row_s, starts, zm, zp,
                     magg_out, pagg_out,
                     sbuf, idx, mbuf, cbuf, acc, acc16):
  c = lax.axis_index("c")
  s = lax.axis_index("s")
  w = s * NC + c
  nw0 = w * NPT

  pltpu.sync_copy(zm, acc)
  pltpu.sync_copy(zp, acc16)
  pltpu.sync_copy(starts, sbuf.at[pl.ds(0, 40)])
  start = sbuf[pl.ds(w, 16)][0]
  end = sbuf[pl.ds(w + 1, 16)][0]
  astart = (start // 8) * 8
  nch = (end - astart + CH - 1) // CH

  def chunk(k, carry):
    lo_k = astart + k * CH
    off = pl.multiple_of(jnp.minimum(lo_k, NE - CH), 8)
    pltpu.sync_copy(row_s.at[pl.ds(off, CH)], idx.at[pl.ds(0, CH)])
    pltpu.sync_copy(m.at[pl.ds(off, CH)], mbuf)
    pltpu.sync_copy(cwu.at[pl.ds(off * 16, CH * 16)], cbuf)
    lo = jnp.maximum(start, lo_k)

    def edge(e, cc):
      eg = off + e
      @pl.when(jnp.logical_and(eg >= lo, eg < end))
      def _():
        rel = idx[pl.ds(e, 16)][0] - nw0
        for j in range(HD // 16):
          sl = pl.ds(j * 16, 16)
          acc[rel, sl] += mbuf[e, sl]
        sl16 = pl.ds(rel * 16, 16)
        acc16[sl16] += cbuf[pl.ds(e * 16, 16)]
      return cc

    lax.fori_loop(0, CH, edge, 0)
    return carry

  lax.fori_loop(0, nch, chunk, 0)

  @pl.when(w < NW - 1)
  def _():
    pltpu.sync_copy(acc, magg_out.at[pl.ds(nw0, NPT)])
    pltpu.sync_copy(acc16, pagg_out.at[pl.ds(nw0 * 16, NPT * 16)])
  @pl.when(w == NW - 1)
  def _():
    pltpu.sync_copy(acc.at[pl.ds(0, NPT_LAST)],
                    magg_out.at[pl.ds(nw0, NPT_LAST)])
    pltpu.sync_copy(acc16.at[pl.ds(0, NPT_LAST * 16)],
                    pagg_out.at[pl.ds(nw0 * 16, NPT_LAST * 16)])


@functools.cache
def _sc_scatter():
  return pl.kernel(
      _sc_scatter_body,
      out_type=(
          jax.ShapeDtypeStruct((NN, HD), jnp.float32),
          jax.ShapeDtypeStruct((NN * 16,), jnp.float32),
      ),
      mesh=_sc_mesh(),
      compiler_params=pltpu.CompilerParams(needs_layout_passes=False),
      scratch_types=[
          pltpu.VMEM((56,), jnp.int32),
          pltpu.VMEM((CH + 16,), jnp.int32),
          pltpu.VMEM((CH, HD), jnp.float32),
          pltpu.VMEM((CH * 16,), jnp.float32),
          pltpu.VMEM((NPT, HD), jnp.float32),
          pltpu.VMEM((NPT * 16,), jnp.float32),
      ],
  )


def _silu(x):
  return x * jax.nn.sigmoid(x)


def _mm(a, b):
  return jnp.dot(a, b, preferred_element_type=jnp.float32)


def _temb_body(t_ref, freqs_ref, wt1_ref, bt1_ref, wt2_ref, bt2_ref, out_ref):
  phase = t_ref[...] * freqs_ref[...]
  lane = lax.broadcasted_iota(jnp.int32, phase.shape, 1)
  se = jnp.where(lane < HD // 2, jnp.sin(phase), jnp.cos(phase))
  h1 = _silu(_mm(se, wt1_ref[...]) + bt1_ref[...])
  out_ref[...] = _mm(h1, wt2_ref[...]) + bt2_ref[...]


def _temb(t2d, freqs2, p):
  return pl.pallas_call(
      _temb_body,
      out_shape=jax.ShapeDtypeStruct((NB, HD), jnp.float32),
  )(t2d, freqs2, p['Wt1'], p['bt1'].reshape(1, HD), p['Wt2'],
    p['bt2'].reshape(1, HD))


BN = 2000


def _embed_body(at_ref, batch_ref, emb_ref, temb_ref, h_ref, tn_ref):
  at = at_ref[...]
  oh_a = (at == lax.broadcasted_iota(jnp.int32, (BN, 16), 1)).astype(jnp.float32)
  h_ref[...] = _mm(oh_a, emb_ref[...])
  bt = batch_ref[...]
  oh_b = (bt == lax.broadcasted_iota(jnp.int32, (BN, NB), 1)).astype(jnp.float32)
  tn_ref[...] = _mm(oh_b, temb_ref[...])


def _embed(at2d, batch2d, emb16, t_emb):
  grid = NN // BN
  return pl.pallas_call(
      _embed_body,
      grid=(grid,),
      in_specs=[
          pl.BlockSpec((BN, 1), lambda i: (i, 0)),
          pl.BlockSpec((BN, 1), lambda i: (i, 0)),
          pl.BlockSpec((16, HD), lambda i: (0, 0)),
          pl.BlockSpec((NB, HD), lambda i: (0, 0)),
      ],
      out_specs=[
          pl.BlockSpec((BN, HD), lambda i: (i, 0)),
          pl.BlockSpec((BN, HD), lambda i: (i, 0)),
      ],
      out_shape=[
          jax.ShapeDtypeStruct((NN, HD), jnp.float32),
          jax.ShapeDtypeStruct((NN, HD), jnp.float32),
      ],
  )(at2d, batch2d, emb16, t_emb)


def _prep_body(h_ref, tn_ref, whr_ref, whc_ref, wt_ref, be1_ref, a_ref, b_ref):
  h = h_ref[...]
  a_ref[...] = _mm(h, whr_ref[...]) + _mm(tn_ref[...], wt_ref[...]) + be1_ref[...]
  b_ref[...] = _mm(h, whc_ref[...])


def _prep(h, t_node, whr, whc, wt, be1l):
  grid = NN // BN
  blk = pl.BlockSpec((BN, HD), lambda i: (i, 0))
  wblk = pl.BlockSpec((HD, HD), lambda i: (0, 0))
  vblk = pl.BlockSpec((1, HD), lambda i: (0, 0))
  return pl.pallas_call(
      _prep_body,
      grid=(grid,),
      in_specs=[blk, blk, wblk, wblk, wblk, vblk],
      out_specs=[blk, blk],
      out_shape=[
          jax.ShapeDtypeStruct((NN, HD), jnp.float32),
          jax.ShapeDtypeStruct((NN, HD), jnp.float32),
      ],
  )(h, t_node, whr, whc, wt, be1l)


BE = 2000


def _edge_body(ar_ref, br_ref, du_ref, wd_ref, we2_ref, be2_ref,
               wc1_ref, bc1_ref, wc2_ref, m_ref, cwu_ref):
  du = du_ref[...]
  dist = du[:, 3:4]
  pre = ar_ref[...] + br_ref[...] + dist * wd_ref[...]
  m1 = _silu(pre)
  m = _silu(_mm(m1, we2_ref[...]) + be2_ref[...])
  m_ref[...] = m
  cwv = _silu(_mm(m, wc1_ref[...]) + bc1_ref[...])
  cw = jnp.sum(cwv * wc2_ref[...], axis=-1, keepdims=True)
  lane = lax.broadcasted_iota(jnp.int32, (BE, 16), 1)
  ux, uy, uz = du[:, 0:1], du[:, 1:2], du[:, 2:3]
  unit_l = jnp.where(lane == 0, ux, 0.0)
  unit_l = jnp.where(lane == 1, uy, unit_l)
  unit_l = jnp.where(lane == 2, uz, unit_l)
  cwu_ref[...] = cw * unit_l


def _edge(ar, br, du, wd, we2, be2l, wc1, bc1l, wc2row):
  grid = NE // BE
  eblk = pl.BlockSpec((BE, HD), lambda i: (i, 0))
  dblk = pl.BlockSpec((BE, 4), lambda i: (i, 0))
  pblk = pl.BlockSpec((BE, 16), lambda i: (i, 0))
  wblk = pl.BlockSpec((HD, HD), lambda i: (0, 0))
  vblk = pl.BlockSpec((1, HD), lambda i: (0, 0))
  return pl.pallas_call(
      _edge_body,
      grid=(grid,),
      in_specs=[eblk, eblk, dblk, vblk, wblk, vblk, wblk, vblk, vblk],
      out_specs=[eblk, pblk],
      out_shape=[
          jax.ShapeDtypeStruct((NE, HD), jnp.float32),
          jax.ShapeDtypeStruct((NE, 16), jnp.float32),
      ],
  )(ar, br, du, wd, we2, be2l, wc1, bc1l, wc2row)


def _node_body(h_ref, msg_ref, tn_ref, p4_ref, pd_ref,
               wnh_ref, wnm_ref, wnt_ref, bn1_ref, wn2_ref, bn2_ref,
               gam_ref, bet_ref, hn_ref, pn_ref):
  h = h_ref[...]
  msg = msg_ref[...]
  x = _silu(_mm(h, wnh_ref[...]) + _mm(msg, wnm_ref[...])
            + _mm(tn_ref[...], wnt_ref[...]) + bn1_ref[...])
  hr = h + _mm(x, wn2_ref[...]) + bn2_ref[...]
  mu = jnp.mean(hr, axis=-1, keepdims=True)
  cen = hr - mu
  var = jnp.mean(cen * cen, axis=-1, keepdims=True)
  hn_ref[...] = cen * lax.rsqrt(var + 1e-5) * gam_ref[...] + bet_ref[...]
  pn_ref[...] = p4_ref[...] + pd_ref[...][:, :4]


def _node(h, magg, t_node, pos4, pd, wnh, wnm, wnt, bn1l, wn2, bn2l,
          gaml, betl):
  grid = NN // BN
  blk = pl.BlockSpec((BN, HD), lambda i: (i, 0))
  p4blk = pl.BlockSpec((BN, 4), lambda i: (i, 0))
  p16blk = pl.BlockSpec((BN, 16), lambda i: (i, 0))
  wblk = pl.BlockSpec((HD, HD), lambda i: (0, 0))
  vblk = pl.BlockSpec((1, HD), lambda i: (0, 0))
  return pl.pallas_call(
      _node_body,
      grid=(grid,),
      in_specs=[blk, blk, blk, p4blk, p16blk,
                wblk, wblk, wblk, vblk, wblk, vblk, vblk, vblk],
      out_specs=[blk, p4blk],
      out_shape=[
          jax.ShapeDtypeStruct((NN, HD), jnp.float32),
          jax.ShapeDtypeStruct((NN, 4), jnp.float32),
      ],
  )(h, magg, t_node, pos4, pd,
    wnh, wnm, wnt, bn1l, wn2, bn2l, gaml, betl)


def _head_body(h_ref, w_ref, b_ref, out_ref):
  out_ref[...] = _mm(h_ref[...], w_ref[...]) + b_ref[...]


def _head(h, whead, bhead):
  grid = NN // BN
  return pl.pallas_call(
      _head_body,
      grid=(grid,),
      in_specs=[
          pl.BlockSpec((BN, HD), lambda i: (i, 0)),
          pl.BlockSpec((HD, HD), lambda i: (0, 0)),
          pl.BlockSpec((1, HD), lambda i: (0, 0)),
      ],
      out_specs=pl.BlockSpec((BN, HD), lambda i: (i, 0)),
      out_shape=jax.ShapeDtypeStruct((NN, HD), jnp.float32),
  )(h, whead, bhead)


def kernel(atom_types, pos, edge_index, timesteps, batch, params):
  p = params
  row_u = edge_index[0].astype(jnp.int32)
  perm = jnp.argsort(row_u)
  row32 = row_u[perm]
  col32 = edge_index[1].astype(jnp.int32)[perm]
  bounds = jnp.arange(33, dtype=jnp.int32) * NPT
  starts = jnp.zeros((40,), jnp.int32).at[:33].set(
      jnp.searchsorted(row32, bounds).astype(jnp.int32))
  pos4 = jnp.zeros((NN, 4), jnp.float32).at[:, :3].set(pos)

  half = HD // 2
  freqs = jnp.exp(-math.log(10000.0)
                  * jnp.arange(half, dtype=jnp.float32) / half)
  freqs2 = jnp.concatenate([freqs, freqs]).reshape(1, HD)
  t2d = timesteps.astype(jnp.float32).reshape(NB, 1)
  t_emb = _temb(t2d, freqs2, p)

  emb16 = jnp.zeros((16, HD), jnp.float32).at[:NA + 1].set(p['embed'])
  h, t_node = _embed(atom_types.astype(jnp.int32).reshape(NN, 1),
                     batch.astype(jnp.int32).reshape(NN, 1), emb16, t_emb)

  zm = jnp.zeros((NPT, HD), jnp.float32)
  zp = jnp.zeros((NPT * 16,), jnp.float32)

  for l in range(NL):
    we1 = p['We1'][l]
    whr, whc = we1[:HD], we1[HD:2 * HD]
    wd = we1[2 * HD].reshape(1, HD)
    wt = we1[2 * HD + 1:]
    a_tab, b_tab = _prep(h, t_node, whr, whc, wt, p['be1'][l].reshape(1, HD))
    ar, br, duf = _sc_gather()(a_tab, b_tab, pos4.reshape(NN * 4), row32,
                               col32)
    m, cwu = _edge(ar, br, duf.reshape(NE, 4), wd, p['We2'][l],
                   p['be2'][l].reshape(1, HD), p['Wc1'][l],
                   p['bc1'][l].reshape(1, HD), p['Wc2'][l].reshape(1, HD))
    magg, paggf = _sc_scatter()(m, cwu.reshape(NE * 16), row32, starts,
                                zm, zp)
    pagg = paggf.reshape(NN, 16)
    wn1 = p['Wn1'][l]
    h, pos4 = _node(h, magg, t_node, pos4, pagg,
                    wn1[:HD], wn1[HD:2 * HD], wn1[2 * HD:],
                    p['bn1'][l].reshape(1, HD), p['Wn2'][l],
                    p['bn2'][l].reshape(1, HD), p['gamma'][l].reshape(1, HD),
                    p['beta'][l].reshape(1, HD))

  whead = jnp.zeros((HD, HD), jnp.float32)
  whead = whead.at[:, :3].set(p['Wch']).at[:, 3:3 + NA].set(p['Wah'])
  bhead = jnp.zeros((1, HD), jnp.float32)
  bhead = bhead.at[0, :3].set(p['bch']).at[0, 3:3 + NA].set(p['bah'])
  out = _head(h, whead, bhead)
  return out[:, :3], out[:, 3:3 + NA]


# speedup vs baseline: 2.0349x; 1.0610x over previous
import functools
import math

import jax
import jax.numpy as jnp
from jax import lax
from jax.experimental import pallas as pl
from jax.experimental.pallas import tpu as pltpu
from jax.experimental.pallas import tpu_sc as plsc

NN, NE, HD, NB, NL, NA = 10000, 320000, 128, 128, 8, 10
NC, NS = 2, 16
NW = NC * NS
EPW = NE // NW
CH = 200
NCHUNK = EPW // CH
G16 = CH // 16
CHG = 160
NCHG = -(-EPW // CHG)
NPT = (-(-NN // NW) + 7) // 8 * 8
NPT_LAST = NN - (NW - 1) * NPT


@functools.cache
def _sc_mesh():
  return plsc.VectorSubcoreMesh(
      core_axis_name="c", subcore_axis_name="s", num_cores=NC, num_subcores=NS)


def _rsqrt16(x):
  i = plsc.bitcast(x, jnp.int32)
  i = jnp.int32(0x5F3759DF) - lax.shift_right_logical(i, 1)
  y = plsc.bitcast(i, jnp.float32)
  for _ in range(3):
    y = y * (1.5 - 0.5 * x * y * y)
  return y


def _sc_gather_body(a_tab, b_tab, pos4, row, col,
                    ar_out, br_out, du_out,
                    idxr0, idxc0, abuf0, bbuf0, dubuf0,
                    idxr1, idxc1, abuf1, bbuf1, dubuf1,
                    pos_v,
                    sga0, sgb0, swa0, swb0, swd0,
                    sga1, sgb1, swa1, swb1, swd1):
  c = lax.axis_index("c")
  s = lax.axis_index("s")
  base = (s * NC + c) * EPW

  pltpu.sync_copy(pos4, pos_v)
  bufs = ((idxr0, idxc0, abuf0, bbuf0, dubuf0, sga0, sgb0, swa0, swb0, swd0),
          (idxr1, idxc1, abuf1, bbuf1, dubuf1, sga1, sgb1, swa1, swb1, swd1))

  def off_of(k):
    return pl.multiple_of(
        jnp.minimum(base + k * CHG, base + EPW - CHG), 8)

  def fire(p, k):
    idxr, idxc, abuf, bbuf = bufs[p][:4]
    sga, sgb = bufs[p][5], bufs[p][6]
    off = off_of(k)
    pltpu.sync_copy(row.at[pl.ds(off, CHG)], idxr)
    pltpu.sync_copy(col.at[pl.ds(off, CHG)], idxc)
    pltpu.async_copy(a_tab.at[idxr], abuf, sga)
    pltpu.async_copy(b_tab.at[idxc], bbuf, sgb)

  def finish(p, k):
    idxr, idxc, abuf, bbuf, dubuf, sga, sgb, swa, swb, swd = bufs[p]
    off = off_of(k)
    lanes = lax.iota(jnp.int32, 16)
    for g in range(CHG // 16):
      r16 = idxr[pl.ds(g * 16, 16)]
      c16 = idxc[pl.ds(g * 16, 16)]
      d = []
      for comp in range(3):
        pr = plsc.load_gather(pos_v, [r16 * 4 + comp])
        qr = plsc.load_gather(pos_v, [c16 * 4 + comp])
        d.append(pr - qr)
      d2 = d[0] * d[0] + d[1] * d[1] + d[2] * d[2] + 1e-8
      y = _rsqrt16(d2)
      dist = d2 * y
      inv = 1.0 / (dist + 1e-8)
      e16 = g * 16 + lanes
      for comp in range(3):
        plsc.store_scatter(dubuf, [e16 * 4 + comp], d[comp] * inv)
      plsc.store_scatter(dubuf, [e16 * 4 + 3], dist)
    pltpu.make_async_copy(a_tab.at[idxr], abuf, sga).wait()
    pltpu.make_async_copy(b_tab.at[idxc], bbuf, sgb).wait()
    pltpu.async_copy(abuf, ar_out.at[pl.ds(off, CHG)], swa)
    pltpu.async_copy(bbuf, br_out.at[pl.ds(off, CHG)], swb)
    pltpu.async_copy(dubuf, du_out.at[pl.ds(off * 4, CHG * 4)], swd)

  def wait_writes(p):
    abuf, bbuf, dubuf = bufs[p][2], bufs[p][3], bufs[p][4]
    swa, swb, swd = bufs[p][7], bufs[p][8], bufs[p][9]
    pltpu.make_async_copy(abuf, ar_out.at[pl.ds(0, CHG)], swa).wait()
    pltpu.make_async_copy(bbuf, br_out.at[pl.ds(0, CHG)], swb).wait()
    pltpu.make_async_copy(dubuf, du_out.at[pl.ds(0, CHG * 4)], swd).wait()

  def body(j, carry):
    @pl.when(j > 0)
    def _():
      wait_writes(0)
      wait_writes(1)
    fire(0, 2 * j)
    fire(1, 2 * j + 1)
    finish(0, 2 * j)
    finish(1, 2 * j + 1)
    return carry

  lax.fori_loop(0, (NCHG + 1) // 2, body, 0)
  wait_writes(0)
  wait_writes(1)


@functools.cache
def _sc_gather():
  return pl.kernel(
      _sc_gather_body,
      out_type=(
          jax.ShapeDtypeStruct((NE, HD), jnp.float32),
          jax.ShapeDtypeStruct((NE, HD), jnp.float32),
          jax.ShapeDtypeStruct((NE * 4,), jnp.float32),
      ),
      mesh=_sc_mesh(),
      compiler_params=pltpu.CompilerParams(needs_layout_passes=False),
      scratch_types=[
          pltpu.VMEM((CHG,), jnp.int32),
          pltpu.VMEM((CHG,), jnp.int32),
          pltpu.VMEM((CHG, HD), jnp.float32),
          pltpu.VMEM((CHG, HD), jnp.float32),
          pltpu.VMEM((CHG * 4,), jnp.float32),
          pltpu.VMEM((CHG,), jnp.int32),
          pltpu.VMEM((CHG,), jnp.int32),
          pltpu.VMEM((CHG, HD), jnp.float32),
          pltpu.VMEM((CHG, HD), jnp.float32),
          pltpu.VMEM((CHG * 4,), jnp.float32),
          pltpu.VMEM((NN * 4,), jnp.float32),
      ] + [pltpu.SemaphoreType.DMA] * 10,
  )


def _sc_scatter_body(m, cwu, row_s, starts, zm, zp,
                     magg_out, pagg_out,
                     sbuf, idx0, mbuf0, cbuf0, idx1, mbuf1, cbuf1, acc, acc16,
                     si0, sm0, sc0, si1, sm1, sc1):
  c = lax.axis_index("c")
  s = lax.axis_index("s")
  w = s * NC + c
  nw0 = w * NPT

  pltpu.sync_copy(zm, acc)
  pltpu.sync_copy(zp, acc16)
  pltpu.sync_copy(starts, sbuf.at[pl.ds(0, 40)])
  start = sbuf[pl.ds(w, 16)][0]
  end = sbuf[pl.ds(w + 1, 16)][0]
  astart = (start // 8) * 8
  nch = (end - astart + CH - 1) // CH
  bufs = ((idx0, mbuf0, cbuf0, si0, sm0, sc0),
          (idx1, mbuf1, cbuf1, si1, sm1, sc1))

  def off_of(k):
    return pl.multiple_of(jnp.minimum(astart + k * CH, NE - CH), 8)

  def fire(p, k):
    idx, mbuf, cbuf, si, sm, sc = bufs[p]
    off = off_of(k)
    pltpu.async_copy(row_s.at[pl.ds(off, CH)], idx.at[pl.ds(0, CH)], si)
    pltpu.async_copy(m.at[pl.ds(off, CH)], mbuf, sm)
    pltpu.async_copy(cwu.at[pl.ds(off * 16, CH * 16)], cbuf, sc)

  def process(p, k):
    idx, mbuf, cbuf, si, sm, sc = bufs[p]
    off = off_of(k)
    pltpu.make_async_copy(row_s.at[pl.ds(0, CH)], idx.at[pl.ds(0, CH)],
                          si).wait()
    pltpu.make_async_copy(m.at[pl.ds(0, CH)], mbuf, sm).wait()
    pltpu.make_async_copy(cwu.at[pl.ds(0, CH * 16)], cbuf, sc).wait()
    lo = jnp.maximum(start, astart + k * CH)

    def edge(e, cc):
      eg = off + e
      @pl.when(jnp.logical_and(eg >= lo, eg < end))
      def _():
        rel = idx[pl.ds(e, 16)][0] - nw0
        for j in range(HD // 16):
          sl = pl.ds(j * 16, 16)
          acc[rel, sl] += mbuf[e, sl]
        sl16 = pl.ds(rel * 16, 16)
        acc16[sl16] += cbuf[pl.ds(e * 16, 16)]
      return cc

    lax.fori_loop(0, CH, edge, 0)

  def body(j, carry):
    k0 = 2 * j
    fire(0, k0)
    fire(1, k0 + 1)
    process(0, k0)
    process(1, k0 + 1)
    return carry

  lax.fori_loop(0, (nch + 1) // 2, body, 0)

  @pl.when(w < NW - 1)
  def _():
    pltpu.sync_copy(acc, magg_out.at[pl.ds(nw0, NPT)])
    pltpu.sync_copy(acc16, pagg_out.at[pl.ds(nw0 * 16, NPT * 16)])
  @pl.when(w == NW - 1)
  def _():
    pltpu.sync_copy(acc.at[pl.ds(0, NPT_LAST)],
                    magg_out.at[pl.ds(nw0, NPT_LAST)])
    pltpu.sync_copy(acc16.at[pl.ds(0, NPT_LAST * 16)],
                    pagg_out.at[pl.ds(nw0 * 16, NPT_LAST * 16)])


@functools.cache
def _sc_scatter():
  return pl.kernel(
      _sc_scatter_body,
      out_type=(
          jax.ShapeDtypeStruct((NN, HD), jnp.float32),
          jax.ShapeDtypeStruct((NN * 16,), jnp.float32),
      ),
      mesh=_sc_mesh(),
      compiler_params=pltpu.CompilerParams(needs_layout_passes=False),
      scratch_types=[
          pltpu.VMEM((56,), jnp.int32),
          pltpu.VMEM((CH + 16,), jnp.int32),
          pltpu.VMEM((CH, HD), jnp.float32),
          pltpu.VMEM((CH * 16,), jnp.float32),
          pltpu.VMEM((CH + 16,), jnp.int32),
          pltpu.VMEM((CH, HD), jnp.float32),
          pltpu.VMEM((CH * 16,), jnp.float32),
          pltpu.VMEM((NPT, HD), jnp.float32),
          pltpu.VMEM((NPT * 16,), jnp.float32),
      ] + [pltpu.SemaphoreType.DMA] * 6,
  )


def _silu(x):
  return x * jax.nn.sigmoid(x)


def _mm(a, b):
  return jnp.dot(a, b, preferred_element_type=jnp.float32)


def _temb_body(t_ref, freqs_ref, wt1_ref, bt1_ref, wt2_ref, bt2_ref, out_ref):
  phase = t_ref[...] * freqs_ref[...]
  lane = lax.broadcasted_iota(jnp.int32, phase.shape, 1)
  se = jnp.where(lane < HD // 2, jnp.sin(phase), jnp.cos(phase))
  h1 = _silu(_mm(se, wt1_ref[...]) + bt1_ref[...])
  out_ref[...] = _mm(h1, wt2_ref[...]) + bt2_ref[...]


def _temb(t2d, freqs2, p):
  return pl.pallas_call(
      _temb_body,
      out_shape=jax.ShapeDtypeStruct((NB, HD), jnp.float32),
  )(t2d, freqs2, p['Wt1'], p['bt1'].reshape(1, HD), p['Wt2'],
    p['bt2'].reshape(1, HD))


BN = 2000


def _embed_body(at_ref, batch_ref, emb_ref, temb_ref, h_ref, tn_ref):
  at = at_ref[...]
  oh_a = (at == lax.broadcasted_iota(jnp.int32, (BN, 16), 1)).astype(jnp.float32)
  h_ref[...] = _mm(oh_a, emb_ref[...])
  bt = batch_ref[...]
  oh_b = (bt == lax.broadcasted_iota(jnp.int32, (BN, NB), 1)).astype(jnp.float32)
  tn_ref[...] = _mm(oh_b, temb_ref[...])


def _embed(at2d, batch2d, emb16, t_emb):
  grid = NN // BN
  return pl.pallas_call(
      _embed_body,
      grid=(grid,),
      in_specs=[
          pl.BlockSpec((BN, 1), lambda i: (i, 0)),
          pl.BlockSpec((BN, 1), lambda i: (i, 0)),
          pl.BlockSpec((16, HD), lambda i: (0, 0)),
          pl.BlockSpec((NB, HD), lambda i: (0, 0)),
      ],
      out_specs=[
          pl.BlockSpec((BN, HD), lambda i: (i, 0)),
          pl.BlockSpec((BN, HD), lambda i: (i, 0)),
      ],
      out_shape=[
          jax.ShapeDtypeStruct((NN, HD), jnp.float32),
          jax.ShapeDtypeStruct((NN, HD), jnp.float32),
      ],
  )(at2d, batch2d, emb16, t_emb)


def _prep_body(h_ref, tn_ref, whr_ref, whc_ref, wt_ref, be1_ref, a_ref, b_ref):
  h = h_ref[...]
  a_ref[...] = _mm(h, whr_ref[...]) + _mm(tn_ref[...], wt_ref[...]) + be1_ref[...]
  b_ref[...] = _mm(h, whc_ref[...])


def _prep(h, t_node, whr, whc, wt, be1l):
  grid = NN // BN
  blk = pl.BlockSpec((BN, HD), lambda i: (i, 0))
  wblk = pl.BlockSpec((HD, HD), lambda i: (0, 0))
  vblk = pl.BlockSpec((1, HD), lambda i: (0, 0))
  return pl.pallas_call(
      _prep_body,
      grid=(grid,),
      in_specs=[blk, blk, wblk, wblk, wblk, vblk],
      out_specs=[blk, blk],
      out_shape=[
          jax.ShapeDtypeStruct((NN, HD), jnp.float32),
          jax.ShapeDtypeStruct((NN, HD), jnp.float32),
      ],
  )(h, t_node, whr, whc, wt, be1l)


BE = 2000


def _edge_body(ar_ref, br_ref, du_ref, wd_ref, we2_ref, be2_ref,
               wc1_ref, bc1_ref, wc2_ref, m_ref, cwu_ref):
  du = du_ref[...]
  dist = du[:, 3:4]
  pre = ar_ref[...] + br_ref[...] + dist * wd_ref[...]
  m1 = _silu(pre)
  m = _silu(_mm(m1, we2_ref[...]) + be2_ref[...])
  m_ref[...] = m
  cwv = _silu(_mm(m, wc1_ref[...]) + bc1_ref[...])
  cw = jnp.sum(cwv * wc2_ref[...], axis=-1, keepdims=True)
  lane = lax.broadcasted_iota(jnp.int32, (BE, 16), 1)
  ux, uy, uz = du[:, 0:1], du[:, 1:2], du[:, 2:3]
  unit_l = jnp.where(lane == 0, ux, 0.0)
  unit_l = jnp.where(lane == 1, uy, unit_l)
  unit_l = jnp.where(lane == 2, uz, unit_l)
  cwu_ref[...] = cw * unit_l


def _edge(ar, br, du, wd, we2, be2l, wc1, bc1l, wc2row):
  grid = NE // BE
  eblk = pl.BlockSpec((BE, HD), lambda i: (i, 0))
  dblk = pl.BlockSpec((BE, 4), lambda i: (i, 0))
  pblk = pl.BlockSpec((BE, 16), lambda i: (i, 0))
  wblk = pl.BlockSpec((HD, HD), lambda i: (0, 0))
  vblk = pl.BlockSpec((1, HD), lambda i: (0, 0))
  return pl.pallas_call(
      _edge_body,
      grid=(grid,),
      in_specs=[eblk, eblk, dblk, vblk, wblk, vblk, wblk, vblk, vblk],
      out_specs=[eblk, pblk],
      out_shape=[
          jax.ShapeDtypeStruct((NE, HD), jnp.float32),
          jax.ShapeDtypeStruct((NE, 16), jnp.float32),
      ],
  )(ar, br, du, wd, we2, be2l, wc1, bc1l, wc2row)


def _node_body(h_ref, msg_ref, tn_ref, p4_ref, pd_ref,
               wnh_ref, wnm_ref, wnt_ref, bn1_ref, wn2_ref, bn2_ref,
               gam_ref, bet_ref, hn_ref, pn_ref):
  h = h_ref[...]
  msg = msg_ref[...]
  x = _silu(_mm(h, wnh_ref[...]) + _mm(msg, wnm_ref[...])
            + _mm(tn_ref[...], wnt_ref[...]) + bn1_ref[...])
  hr = h + _mm(x, wn2_ref[...]) + bn2_ref[...]
  mu = jnp.mean(hr, axis=-1, keepdims=True)
  cen = hr - mu
  var = jnp.mean(cen * cen, axis=-1, keepdims=True)
  hn_ref[...] = cen * lax.rsqrt(var + 1e-5) * gam_ref[...] + bet_ref[...]
  pn_ref[...] = p4_ref[...] + pd_ref[...][:, :4]


def _node(h, magg, t_node, pos4, pd, wnh, wnm, wnt, bn1l, wn2, bn2l,
          gaml, betl):
  grid = NN // BN
  blk = pl.BlockSpec((BN, HD), lambda i: (i, 0))
  p4blk = pl.BlockSpec((BN, 4), lambda i: (i, 0))
  p16blk = pl.BlockSpec((BN, 16), lambda i: (i, 0))
  wblk = pl.BlockSpec((HD, HD), lambda i: (0, 0))
  vblk = pl.BlockSpec((1, HD), lambda i: (0, 0))
  return pl.pallas_call(
      _node_body,
      grid=(grid,),
      in_specs=[blk, blk, blk, p4blk, p16blk,
                wblk, wblk, wblk, vblk, wblk, vblk, vblk, vblk],
      out_specs=[blk, p4blk],
      out_shape=[
          jax.ShapeDtypeStruct((NN, HD), jnp.float32),
          jax.ShapeDtypeStruct((NN, 4), jnp.float32),
      ],
  )(h, magg, t_node, pos4, pd,
    wnh, wnm, wnt, bn1l, wn2, bn2l, gaml, betl)


def _head_body(h_ref, w_ref, b_ref, out_ref):
  out_ref[...] = _mm(h_ref[...], w_ref[...]) + b_ref[...]


def _head(h, whead, bhead):
  grid = NN // BN
  return pl.pallas_call(
      _head_body,
      grid=(grid,),
      in_specs=[
          pl.BlockSpec((BN, HD), lambda i: (i, 0)),
          pl.BlockSpec((HD, HD), lambda i: (0, 0)),
          pl.BlockSpec((1, HD), lambda i: (0, 0)),
      ],
      out_specs=pl.BlockSpec((BN, HD), lambda i: (i, 0)),
      out_shape=jax.ShapeDtypeStruct((NN, HD), jnp.float32),
  )(h, whead, bhead)


def kernel(atom_types, pos, edge_index, timesteps, batch, params):
  p = params
  row_u = edge_index[0].astype(jnp.int32)
  perm = jnp.argsort(row_u)
  row32 = row_u[perm]
  col32 = edge_index[1].astype(jnp.int32)[perm]
  bounds = jnp.arange(33, dtype=jnp.int32) * NPT
  starts = jnp.zeros((40,), jnp.int32).at[:33].set(
      jnp.searchsorted(row32, bounds).astype(jnp.int32))
  pos4 = jnp.zeros((NN, 4), jnp.float32).at[:, :3].set(pos)

  half = HD // 2
  freqs = jnp.exp(-math.log(10000.0)
                  * jnp.arange(half, dtype=jnp.float32) / half)
  freqs2 = jnp.concatenate([freqs, freqs]).reshape(1, HD)
  t2d = timesteps.astype(jnp.float32).reshape(NB, 1)
  t_emb = _temb(t2d, freqs2, p)

  emb16 = jnp.zeros((16, HD), jnp.float32).at[:NA + 1].set(p['embed'])
  h, t_node = _embed(atom_types.astype(jnp.int32).reshape(NN, 1),
                     batch.astype(jnp.int32).reshape(NN, 1), emb16, t_emb)

  zm = jnp.zeros((NPT, HD), jnp.float32)
  zp = jnp.zeros((NPT * 16,), jnp.float32)

  for l in range(NL):
    we1 = p['We1'][l]
    whr, whc = we1[:HD], we1[HD:2 * HD]
    wd = we1[2 * HD].reshape(1, HD)
    wt = we1[2 * HD + 1:]
    a_tab, b_tab = _prep(h, t_node, whr, whc, wt, p['be1'][l].reshape(1, HD))
    ar, br, duf = _sc_gather()(a_tab, b_tab, pos4.reshape(NN * 4), row32,
                               col32)
    m, cwu = _edge(ar, br, duf.reshape(NE, 4), wd, p['We2'][l],
                   p['be2'][l].reshape(1, HD), p['Wc1'][l],
                   p['bc1'][l].reshape(1, HD), p['Wc2'][l].reshape(1, HD))
    magg, paggf = _sc_scatter()(m, cwu.reshape(NE * 16), row32, starts,
                                zm, zp)
    pagg = paggf.reshape(NN, 16)
    wn1 = p['Wn1'][l]
    h, pos4 = _node(h, magg, t_node, pos4, pagg,
                    wn1[:HD], wn1[HD:2 * HD], wn1[2 * HD:],
                    p['bn1'][l].reshape(1, HD), p['Wn2'][l],
                    p['bn2'][l].reshape(1, HD), p['gamma'][l].reshape(1, HD),
                    p['beta'][l].reshape(1, HD))

  whead = jnp.zeros((HD, HD), jnp.float32)
  whead = whead.at[:, :3].set(p['Wch']).at[:, 3:3 + NA].set(p['Wah'])
  bhead = jnp.zeros((1, HD), jnp.float32)
  bhead = bhead.at[0, :3].set(p['bch']).at[0, 3:3 + NA].set(p['bah'])
  out = _head(h, whead, bhead)
  return out[:, :3], out[:, 3:3 + NA]


# speedup vs baseline: 2.1238x; 1.0437x over previous
import functools
import math

import jax
import jax.numpy as jnp
from jax import lax
from jax.experimental import pallas as pl
from jax.experimental.pallas import tpu as pltpu
from jax.experimental.pallas import tpu_sc as plsc

NN, NE, HD, NB, NL, NA = 10000, 320000, 128, 128, 8, 10
NC, NS = 2, 16
NW = NC * NS
EPW = NE // NW
CH = 200
NCHUNK = EPW // CH
G16 = CH // 16
CHG = 160
NCHG = -(-EPW // CHG)
NPT = (-(-NN // NW) + 7) // 8 * 8
NPT_LAST = NN - (NW - 1) * NPT


@functools.cache
def _sc_mesh():
  return plsc.VectorSubcoreMesh(
      core_axis_name="c", subcore_axis_name="s", num_cores=NC, num_subcores=NS)


def _rsqrt16(x):
  i = plsc.bitcast(x, jnp.int32)
  i = jnp.int32(0x5F3759DF) - lax.shift_right_logical(i, 1)
  y = plsc.bitcast(i, jnp.float32)
  for _ in range(3):
    y = y * (1.5 - 0.5 * x * y * y)
  return y


def _sc_gather_body(a_tab, b_tab, pos4, row, col,
                    ar_out, br_out, du_out,
                    idxr0, idxc0, abuf0, bbuf0, dubuf0,
                    idxr1, idxc1, abuf1, bbuf1, dubuf1,
                    pos_v,
                    sga0, sgb0, swa0, swb0, swd0,
                    sga1, sgb1, swa1, swb1, swd1):
  c = lax.axis_index("c")
  s = lax.axis_index("s")
  base = (s * NC + c) * EPW

  pltpu.sync_copy(pos4, pos_v)
  bufs = ((idxr0, idxc0, abuf0, bbuf0, dubuf0, sga0, sgb0, swa0, swb0, swd0),
          (idxr1, idxc1, abuf1, bbuf1, dubuf1, sga1, sgb1, swa1, swb1, swd1))

  def off_of(k):
    return pl.multiple_of(
        jnp.minimum(base + k * CHG, base + EPW - CHG), 8)

  def fire(p, k):
    idxr, idxc, abuf, bbuf = bufs[p][:4]
    sga, sgb = bufs[p][5], bufs[p][6]
    off = off_of(k)
    pltpu.sync_copy(row.at[pl.ds(off, CHG)], idxr)
    pltpu.sync_copy(col.at[pl.ds(off, CHG)], idxc)
    pltpu.async_copy(a_tab.at[idxr], abuf, sga)
    pltpu.async_copy(b_tab.at[idxc], bbuf, sgb)

  def finish(p, k):
    idxr, idxc, abuf, bbuf, dubuf, sga, sgb, swa, swb, swd = bufs[p]
    off = off_of(k)
    lanes = lax.iota(jnp.int32, 16)
    for g in range(CHG // 16):
      r16 = idxr[pl.ds(g * 16, 16)]
      c16 = idxc[pl.ds(g * 16, 16)]
      d = []
      for comp in range(3):
        pr = plsc.load_gather(pos_v, [r16 * 4 + comp])
        qr = plsc.load_gather(pos_v, [c16 * 4 + comp])
        d.append(pr - qr)
      d2 = d[0] * d[0] + d[1] * d[1] + d[2] * d[2] + 1e-8
      y = _rsqrt16(d2)
      dist = d2 * y
      inv = 1.0 / (dist + 1e-8)
      e16 = g * 16 + lanes
      for comp in range(3):
        plsc.store_scatter(dubuf, [e16 * 4 + comp], d[comp] * inv)
      plsc.store_scatter(dubuf, [e16 * 4 + 3], dist)
    pltpu.make_async_copy(a_tab.at[idxr], abuf, sga).wait()
    pltpu.make_async_copy(b_tab.at[idxc], bbuf, sgb).wait()
    pltpu.async_copy(abuf, ar_out.at[pl.ds(off, CHG)], swa)
    pltpu.async_copy(bbuf, br_out.at[pl.ds(off, CHG)], swb)
    pltpu.async_copy(dubuf, du_out.at[pl.ds(off * 4, CHG * 4)], swd)

  def wait_writes(p):
    abuf, bbuf, dubuf = bufs[p][2], bufs[p][3], bufs[p][4]
    swa, swb, swd = bufs[p][7], bufs[p][8], bufs[p][9]
    pltpu.make_async_copy(abuf, ar_out.at[pl.ds(0, CHG)], swa).wait()
    pltpu.make_async_copy(bbuf, br_out.at[pl.ds(0, CHG)], swb).wait()
    pltpu.make_async_copy(dubuf, du_out.at[pl.ds(0, CHG * 4)], swd).wait()

  def body(j, carry):
    @pl.when(j > 0)
    def _():
      wait_writes(0)
      wait_writes(1)
    fire(0, 2 * j)
    fire(1, 2 * j + 1)
    finish(0, 2 * j)
    finish(1, 2 * j + 1)
    return carry

  lax.fori_loop(0, (NCHG + 1) // 2, body, 0)
  wait_writes(0)
  wait_writes(1)


@functools.cache
def _sc_gather():
  return pl.kernel(
      _sc_gather_body,
      out_type=(
          jax.ShapeDtypeStruct((NE, HD), jnp.float32),
          jax.ShapeDtypeStruct((NE, HD), jnp.float32),
          jax.ShapeDtypeStruct((NE * 4,), jnp.float32),
      ),
      mesh=_sc_mesh(),
      compiler_params=pltpu.CompilerParams(needs_layout_passes=False),
      scratch_types=[
          pltpu.VMEM((CHG,), jnp.int32),
          pltpu.VMEM((CHG,), jnp.int32),
          pltpu.VMEM((CHG, HD), jnp.float32),
          pltpu.VMEM((CHG, HD), jnp.float32),
          pltpu.VMEM((CHG * 4,), jnp.float32),
          pltpu.VMEM((CHG,), jnp.int32),
          pltpu.VMEM((CHG,), jnp.int32),
          pltpu.VMEM((CHG, HD), jnp.float32),
          pltpu.VMEM((CHG, HD), jnp.float32),
          pltpu.VMEM((CHG * 4,), jnp.float32),
          pltpu.VMEM((NN * 4,), jnp.float32),
      ] + [pltpu.SemaphoreType.DMA] * 10,
  )


def _sc_scatter_body(m, cwu, row_s, starts, zm, zp,
                     magg_out, pagg_out,
                     sbuf, idx0, mbuf0, cbuf0, idx1, mbuf1, cbuf1, acc, acc16,
                     si0, sm0, sc0, si1, sm1, sc1):
  c = lax.axis_index("c")
  s = lax.axis_index("s")
  w = s * NC + c
  nw0 = w * NPT

  pltpu.sync_copy(zm, acc)
  pltpu.sync_copy(zp, acc16)
  pltpu.sync_copy(starts, sbuf.at[pl.ds(0, 40)])
  start = sbuf[pl.ds(w, 16)][0]
  end = sbuf[pl.ds(w + 1, 16)][0]
  astart = (start // 8) * 8
  nch = (end - astart + CH - 1) // CH
  bufs = ((idx0, mbuf0, cbuf0, si0, sm0, sc0),
          (idx1, mbuf1, cbuf1, si1, sm1, sc1))

  def off_of(k):
    return pl.multiple_of(jnp.minimum(astart + k * CH, NE - CH), 8)

  def fire(p, k):
    idx, mbuf, cbuf, si, sm, sc = bufs[p]
    off = off_of(k)
    pltpu.async_copy(row_s.at[pl.ds(off, CH)], idx.at[pl.ds(0, CH)], si)
    pltpu.async_copy(m.at[pl.ds(off, CH)], mbuf, sm)
    pltpu.async_copy(cwu.at[pl.ds(off * 16, CH * 16)], cbuf, sc)

  def process(p, k):
    idx, mbuf, cbuf, si, sm, sc = bufs[p]
    off = off_of(k)
    pltpu.make_async_copy(row_s.at[pl.ds(0, CH)], idx.at[pl.ds(0, CH)],
                          si).wait()
    pltpu.make_async_copy(m.at[pl.ds(0, CH)], mbuf, sm).wait()
    pltpu.make_async_copy(cwu.at[pl.ds(0, CH * 16)], cbuf, sc).wait()
    lo = jnp.maximum(start, astart + k * CH)

    def edge(e, cc):
      eg = off + e
      @pl.when(jnp.logical_and(eg >= lo, eg < end))
      def _():
        rel = idx[pl.ds(e, 16)][0] - nw0
        for j in range(HD // 32):
          v32 = mbuf[e, pl.ds(j * 16, 16)]
          lo = plsc.bitcast(lax.shift_left(v32, 16), jnp.float32)
          hi = plsc.bitcast(lax.bitwise_and(v32, jnp.int32(-65536)),
                            jnp.float32)
          acc[rel, pl.ds(j * 16, 16)] += lo
          acc[rel, pl.ds(HD // 2 + j * 16, 16)] += hi
        sl16 = pl.ds(rel * 16, 16)
        acc16[sl16] += cbuf[pl.ds(e * 16, 16)]
      return cc

    lax.fori_loop(0, CH, edge, 0)

  def body(j, carry):
    k0 = 2 * j
    fire(0, k0)
    fire(1, k0 + 1)
    process(0, k0)
    process(1, k0 + 1)
    return carry

  lax.fori_loop(0, (nch + 1) // 2, body, 0)

  @pl.when(w < NW - 1)
  def _():
    pltpu.sync_copy(acc, magg_out.at[pl.ds(nw0, NPT)])
    pltpu.sync_copy(acc16, pagg_out.at[pl.ds(nw0 * 16, NPT * 16)])
  @pl.when(w == NW - 1)
  def _():
    pltpu.sync_copy(acc.at[pl.ds(0, NPT_LAST)],
                    magg_out.at[pl.ds(nw0, NPT_LAST)])
    pltpu.sync_copy(acc16.at[pl.ds(0, NPT_LAST * 16)],
                    pagg_out.at[pl.ds(nw0 * 16, NPT_LAST * 16)])


@functools.cache
def _sc_scatter():
  return pl.kernel(
      _sc_scatter_body,
      out_type=(
          jax.ShapeDtypeStruct((NN, HD), jnp.float32),
          jax.ShapeDtypeStruct((NN * 16,), jnp.float32),
      ),
      mesh=_sc_mesh(),
      compiler_params=pltpu.CompilerParams(needs_layout_passes=False),
      scratch_types=[
          pltpu.VMEM((56,), jnp.int32),
          pltpu.VMEM((CH + 16,), jnp.int32),
          pltpu.VMEM((CH, HD // 2), jnp.int32),
          pltpu.VMEM((CH * 16,), jnp.float32),
          pltpu.VMEM((CH + 16,), jnp.int32),
          pltpu.VMEM((CH, HD // 2), jnp.int32),
          pltpu.VMEM((CH * 16,), jnp.float32),
          pltpu.VMEM((NPT, HD), jnp.float32),
          pltpu.VMEM((NPT * 16,), jnp.float32),
      ] + [pltpu.SemaphoreType.DMA] * 6,
  )


def _silu(x):
  return x * jax.nn.sigmoid(x)


def _mm(a, b):
  return jnp.dot(a, b, preferred_element_type=jnp.float32)


def _temb_body(t_ref, freqs_ref, wt1_ref, bt1_ref, wt2_ref, bt2_ref, out_ref):
  phase = t_ref[...] * freqs_ref[...]
  lane = lax.broadcasted_iota(jnp.int32, phase.shape, 1)
  se = jnp.where(lane < HD // 2, jnp.sin(phase), jnp.cos(phase))
  h1 = _silu(_mm(se, wt1_ref[...]) + bt1_ref[...])
  out_ref[...] = _mm(h1, wt2_ref[...]) + bt2_ref[...]


def _temb(t2d, freqs2, p):
  return pl.pallas_call(
      _temb_body,
      out_shape=jax.ShapeDtypeStruct((NB, HD), jnp.float32),
  )(t2d, freqs2, p['Wt1'], p['bt1'].reshape(1, HD), p['Wt2'],
    p['bt2'].reshape(1, HD))


BN = 2000


def _embed_body(at_ref, batch_ref, emb_ref, temb_ref, h_ref, tn_ref):
  at = at_ref[...]
  oh_a = (at == lax.broadcasted_iota(jnp.int32, (BN, 16), 1)).astype(jnp.float32)
  h_ref[...] = _mm(oh_a, emb_ref[...])
  bt = batch_ref[...]
  oh_b = (bt == lax.broadcasted_iota(jnp.int32, (BN, NB), 1)).astype(jnp.float32)
  tn_ref[...] = _mm(oh_b, temb_ref[...])


def _embed(at2d, batch2d, emb16, t_emb):
  grid = NN // BN
  return pl.pallas_call(
      _embed_body,
      grid=(grid,),
      in_specs=[
          pl.BlockSpec((BN, 1), lambda i: (i, 0)),
          pl.BlockSpec((BN, 1), lambda i: (i, 0)),
          pl.BlockSpec((16, HD), lambda i: (0, 0)),
          pl.BlockSpec((NB, HD), lambda i: (0, 0)),
      ],
      out_specs=[
          pl.BlockSpec((BN, HD), lambda i: (i, 0)),
          pl.BlockSpec((BN, HD), lambda i: (i, 0)),
      ],
      out_shape=[
          jax.ShapeDtypeStruct((NN, HD), jnp.float32),
          jax.ShapeDtypeStruct((NN, HD), jnp.float32),
      ],
  )(at2d, batch2d, emb16, t_emb)


def _prep_body(h_ref, tn_ref, whr_ref, whc_ref, wt_ref, be1_ref, a_ref, b_ref):
  h = h_ref[...]
  a_ref[...] = _mm(h, whr_ref[...]) + _mm(tn_ref[...], wt_ref[...]) + be1_ref[...]
  b_ref[...] = _mm(h, whc_ref[...])


def _prep(h, t_node, whr, whc, wt, be1l):
  grid = NN // BN
  blk = pl.BlockSpec((BN, HD), lambda i: (i, 0))
  wblk = pl.BlockSpec((HD, HD), lambda i: (0, 0))
  vblk = pl.BlockSpec((1, HD), lambda i: (0, 0))
  return pl.pallas_call(
      _prep_body,
      grid=(grid,),
      in_specs=[blk, blk, wblk, wblk, wblk, vblk],
      out_specs=[blk, blk],
      out_shape=[
          jax.ShapeDtypeStruct((NN, HD), jnp.float32),
          jax.ShapeDtypeStruct((NN, HD), jnp.float32),
      ],
  )(h, t_node, whr, whc, wt, be1l)


BE = 2000


def _edge_body(ar_ref, br_ref, du_ref, wd_ref, we2_ref, be2_ref,
               wc1_ref, bc1_ref, wc2_ref, m_ref, cwu_ref):
  du = du_ref[...]
  dist = du[:, 3:4]
  pre = ar_ref[...] + br_ref[...] + dist * wd_ref[...]
  m1 = _silu(pre)
  m = _silu(_mm(m1, we2_ref[...]) + be2_ref[...])
  mlo = lax.shift_right_logical(
      lax.bitcast_convert_type(m[:, :HD // 2], jnp.int32), 16)
  mhi = lax.bitwise_and(lax.bitcast_convert_type(m[:, HD // 2:], jnp.int32),
                        jnp.int32(-65536))
  m_ref[...] = lax.bitwise_or(mlo, mhi)
  cwv = _silu(_mm(m, wc1_ref[...]) + bc1_ref[...])
  cw = jnp.sum(cwv * wc2_ref[...], axis=-1, keepdims=True)
  lane = lax.broadcasted_iota(jnp.int32, (BE, 16), 1)
  ux, uy, uz = du[:, 0:1], du[:, 1:2], du[:, 2:3]
  unit_l = jnp.where(lane == 0, ux, 0.0)
  unit_l = jnp.where(lane == 1, uy, unit_l)
  unit_l = jnp.where(lane == 2, uz, unit_l)
  cwu_ref[...] = cw * unit_l


def _edge(ar, br, du, wd, we2, be2l, wc1, bc1l, wc2row):
  grid = NE // BE
  eblk = pl.BlockSpec((BE, HD), lambda i: (i, 0))
  dblk = pl.BlockSpec((BE, 4), lambda i: (i, 0))
  pblk = pl.BlockSpec((BE, 16), lambda i: (i, 0))
  wblk = pl.BlockSpec((HD, HD), lambda i: (0, 0))
  vblk = pl.BlockSpec((1, HD), lambda i: (0, 0))
  return pl.pallas_call(
      _edge_body,
      grid=(grid,),
      in_specs=[eblk, eblk, dblk, vblk, wblk, vblk, wblk, vblk, vblk],
      out_specs=[pl.BlockSpec((BE, HD // 2), lambda i: (i, 0)), pblk],
      out_shape=[
          jax.ShapeDtypeStruct((NE, HD // 2), jnp.int32),
          jax.ShapeDtypeStruct((NE, 16), jnp.float32),
      ],
  )(ar, br, du, wd, we2, be2l, wc1, bc1l, wc2row)


def _node_body(h_ref, msg_ref, tn_ref, p4_ref, pd_ref,
               wnh_ref, wnm_ref, wnt_ref, bn1_ref, wn2_ref, bn2_ref,
               gam_ref, bet_ref, hn_ref, pn_ref):
  h = h_ref[...]
  msg = msg_ref[...]
  x = _silu(_mm(h, wnh_ref[...]) + _mm(msg, wnm_ref[...])
            + _mm(tn_ref[...], wnt_ref[...]) + bn1_ref[...])
  hr = h + _mm(x, wn2_ref[...]) + bn2_ref[...]
  mu = jnp.mean(hr, axis=-1, keepdims=True)
  cen = hr - mu
  var = jnp.mean(cen * cen, axis=-1, keepdims=True)
  hn_ref[...] = cen * lax.rsqrt(var + 1e-5) * gam_ref[...] + bet_ref[...]
  pn_ref[...] = p4_ref[...] + pd_ref[...][:, :4]


def _node(h, magg, t_node, pos4, pd, wnh, wnm, wnt, bn1l, wn2, bn2l,
          gaml, betl):
  grid = NN // BN
  blk = pl.BlockSpec((BN, HD), lambda i: (i, 0))
  p4blk = pl.BlockSpec((BN, 4), lambda i: (i, 0))
  p16blk = pl.BlockSpec((BN, 16), lambda i: (i, 0))
  wblk = pl.BlockSpec((HD, HD), lambda i: (0, 0))
  vblk = pl.BlockSpec((1, HD), lambda i: (0, 0))
  return pl.pallas_call(
      _node_body,
      grid=(grid,),
      in_specs=[blk, blk, blk, p4blk, p16blk,
                wblk, wblk, wblk, vblk, wblk, vblk, vblk, vblk],
      out_specs=[blk, p4blk],
      out_shape=[
          jax.ShapeDtypeStruct((NN, HD), jnp.float32),
          jax.ShapeDtypeStruct((NN, 4), jnp.float32),
      ],
  )(h, magg, t_node, pos4, pd,
    wnh, wnm, wnt, bn1l, wn2, bn2l, gaml, betl)


def _head_body(h_ref, w_ref, b_ref, out_ref):
  out_ref[...] = _mm(h_ref[...], w_ref[...]) + b_ref[...]


def _head(h, whead, bhead):
  grid = NN // BN
  return pl.pallas_call(
      _head_body,
      grid=(grid,),
      in_specs=[
          pl.BlockSpec((BN, HD), lambda i: (i, 0)),
          pl.BlockSpec((HD, HD), lambda i: (0, 0)),
          pl.BlockSpec((1, HD), lambda i: (0, 0)),
      ],
      out_specs=pl.BlockSpec((BN, HD), lambda i: (i, 0)),
      out_shape=jax.ShapeDtypeStruct((NN, HD), jnp.float32),
  )(h, whead, bhead)


def kernel(atom_types, pos, edge_index, timesteps, batch, params):
  p = params
  row_u = edge_index[0].astype(jnp.int32)
  perm = jnp.argsort(row_u)
  row32 = row_u[perm]
  col32 = edge_index[1].astype(jnp.int32)[perm]
  bounds = jnp.arange(33, dtype=jnp.int32) * NPT
  starts = jnp.zeros((40,), jnp.int32).at[:33].set(
      jnp.searchsorted(row32, bounds).astype(jnp.int32))
  pos4 = jnp.zeros((NN, 4), jnp.float32).at[:, :3].set(pos)

  half = HD // 2
  freqs = jnp.exp(-math.log(10000.0)
                  * jnp.arange(half, dtype=jnp.float32) / half)
  freqs2 = jnp.concatenate([freqs, freqs]).reshape(1, HD)
  t2d = timesteps.astype(jnp.float32).reshape(NB, 1)
  t_emb = _temb(t2d, freqs2, p)

  emb16 = jnp.zeros((16, HD), jnp.float32).at[:NA + 1].set(p['embed'])
  h, t_node = _embed(atom_types.astype(jnp.int32).reshape(NN, 1),
                     batch.astype(jnp.int32).reshape(NN, 1), emb16, t_emb)

  zm = jnp.zeros((NPT, HD), jnp.float32)
  zp = jnp.zeros((NPT * 16,), jnp.float32)

  for l in range(NL):
    we1 = p['We1'][l]
    whr, whc = we1[:HD], we1[HD:2 * HD]
    wd = we1[2 * HD].reshape(1, HD)
    wt = we1[2 * HD + 1:]
    a_tab, b_tab = _prep(h, t_node, whr, whc, wt, p['be1'][l].reshape(1, HD))
    ar, br, duf = _sc_gather()(a_tab, b_tab, pos4.reshape(NN * 4), row32,
                               col32)
    m, cwu = _edge(ar, br, duf.reshape(NE, 4), wd, p['We2'][l],
                   p['be2'][l].reshape(1, HD), p['Wc1'][l],
                   p['bc1'][l].reshape(1, HD), p['Wc2'][l].reshape(1, HD))
    magg, paggf = _sc_scatter()(m, cwu.reshape(NE * 16), row32, starts,
                                zm, zp)
    pagg = paggf.reshape(NN, 16)
    wn1 = p['Wn1'][l]
    h, pos4 = _node(h, magg, t_node, pos4, pagg,
                    wn1[:HD], wn1[HD:2 * HD], wn1[2 * HD:],
                    p['bn1'][l].reshape(1, HD), p['Wn2'][l],
                    p['bn2'][l].reshape(1, HD), p['gamma'][l].reshape(1, HD),
                    p['beta'][l].reshape(1, HD))

  whead = jnp.zeros((HD, HD), jnp.float32)
  whead = whead.at[:, :3].set(p['Wch']).at[:, 3:3 + NA].set(p['Wah'])
  bhead = jnp.zeros((1, HD), jnp.float32)
  bhead = bhead.at[0, :3].set(p['bch']).at[0, 3:3 + NA].set(p['bah'])
  out = _head(h, whead, bhead)
  return out[:, :3], out[:, 3:3 + NA]
